# baseline (speedup 1.0000x reference)
import functools

import jax
import jax.numpy as jnp
from jax import lax
from jax.experimental import pallas as pl
from jax.experimental.pallas import tpu as pltpu
from jax.experimental.pallas import tpu_sc as plsc

_NC = 2
_NS = 16
_NW = _NC * _NS
_LANES = 16


def _sc_gather(table, idx, chunk=12800, unroll=8):
    vocab = table.shape[0]
    n = idx.shape[0]
    n_per_w = n // _NW

    mesh = plsc.VectorSubcoreMesh(core_axis_name="c", subcore_axis_name="s")

    @functools.partial(
        pl.kernel,
        mesh=mesh,
        out_type=jax.ShapeDtypeStruct((n,), jnp.float32),
        scratch_types=[
            pltpu.VMEM((vocab,), jnp.float32),
            pltpu.VMEM((chunk,), jnp.int32),
            pltpu.VMEM((chunk,), jnp.float32),
        ],
        compiler_params=pltpu.CompilerParams(needs_layout_passes=False),
    )
    def gather_kernel(table_hbm, idx_hbm, out_hbm, table_v, idx_v, w_v):
        wid = lax.axis_index("s") * _NC + lax.axis_index("c")
        base = wid * n_per_w
        pltpu.sync_copy(table_hbm, table_v)

        def chunk_body(ci, carry):
            off = pl.multiple_of(base + ci * chunk, 8)
            pltpu.sync_copy(idx_hbm.at[pl.ds(off, chunk)], idx_v)

            def vec_body(i, c2):
                o16 = pl.multiple_of(i * _LANES, _LANES)
                vidx = idx_v[pl.ds(o16, _LANES)]
                w_v[pl.ds(o16, _LANES)] = plsc.load_gather(table_v, [vidx])
                return c2

            lax.fori_loop(0, chunk // _LANES, vec_body, 0, unroll=unroll)
            pltpu.sync_copy(w_v, out_hbm.at[pl.ds(off, chunk)])
            return carry

        lax.fori_loop(0, n_per_w // chunk, chunk_body, 0)

    return gather_kernel(table, idx)


def _scale_rowsum_tc(w, emb2, rows=40960):
    n, d = emb2.shape

    def body(w_ref, x_ref, o_ref):
        ones = jnp.ones((1, x_ref.shape[-1]), jnp.float32)
        s = jax.lax.dot_general(
            ones, x_ref[...],
            dimension_numbers=(((1,), (1,)), ((), ())),
            preferred_element_type=jnp.float32,
        )
        o_ref[...] = w_ref[...] * s[0, :]

    return pl.pallas_call(
        body,
        grid=(n // rows,),
        in_specs=[
            pl.BlockSpec((rows,), lambda i: (i,)),
            pl.BlockSpec((rows, d), lambda i: (i, 0)),
        ],
        out_specs=pl.BlockSpec((rows,), lambda i: (i,)),
        out_shape=jax.ShapeDtypeStruct((n,), jnp.float32),
    )(w, emb2)


def kernel(input_ids, embeddings, u_param):
    b, l = input_ids.shape
    d = embeddings.shape[-1]
    n = b * l
    idx = input_ids.reshape(n).astype(jnp.int32)
    emb2 = embeddings.reshape(n, d)
    w = _sc_gather(u_param.astype(jnp.float32), idx)
    out = _scale_rowsum_tc(w, emb2)
    return out.reshape(b, l)


# speedup vs baseline: 1.0144x; 1.0144x over previous
import functools

import jax
import jax.numpy as jnp
from jax import lax
from jax.experimental import pallas as pl
from jax.experimental.pallas import tpu as pltpu
from jax.experimental.pallas import tpu_sc as plsc

_NC = 2
_NS = 16
_NW = _NC * _NS
_LANES = 16


def _sc_gather(table, idx, chunk=6400, unroll=1):
    vocab = table.shape[0]
    n = idx.shape[0]
    n_per_w = n // _NW

    mesh = plsc.VectorSubcoreMesh(core_axis_name="c", subcore_axis_name="s")

    @functools.partial(
        pl.kernel,
        mesh=mesh,
        out_type=jax.ShapeDtypeStruct((n,), jnp.float32),
        scratch_types=[
            pltpu.VMEM((vocab,), jnp.float32),
            pltpu.VMEM((chunk,), jnp.int32),
            pltpu.VMEM((chunk,), jnp.float32),
        ],
        compiler_params=pltpu.CompilerParams(needs_layout_passes=False),
    )
    def gather_kernel(table_hbm, idx_hbm, out_hbm, table_v, idx_v, w_v):
        wid = lax.axis_index("s") * _NC + lax.axis_index("c")
        base = wid * n_per_w
        pltpu.sync_copy(table_hbm, table_v)

        def chunk_body(ci, carry):
            off = pl.multiple_of(base + ci * chunk, 8)
            pltpu.sync_copy(idx_hbm.at[pl.ds(off, chunk)], idx_v)

            def vec_body(i, c2):
                o16 = pl.multiple_of(i * _LANES, _LANES)
                vidx = idx_v[pl.ds(o16, _LANES)]
                w_v[pl.ds(o16, _LANES)] = plsc.load_gather(table_v, [vidx])
                return c2

            lax.fori_loop(0, chunk // _LANES, vec_body, 0, unroll=unroll)
            pltpu.sync_copy(w_v, out_hbm.at[pl.ds(off, chunk)])
            return carry

        lax.fori_loop(0, n_per_w // chunk, chunk_body, 0)

    return gather_kernel(table, idx)


def _scale_rowsum_tc(w, emb2, rows=40960):
    n, d = emb2.shape

    def body(w_ref, x_ref, o_ref):
        ones = jnp.ones((1, x_ref.shape[-1]), jnp.float32)
        s = jax.lax.dot_general(
            ones, x_ref[...],
            dimension_numbers=(((1,), (1,)), ((), ())),
            preferred_element_type=jnp.float32,
        )
        o_ref[...] = w_ref[...] * s[0, :]

    return pl.pallas_call(
        body,
        grid=(n // rows,),
        in_specs=[
            pl.BlockSpec((rows,), lambda i: (i,)),
            pl.BlockSpec((rows, d), lambda i: (i, 0)),
        ],
        out_specs=pl.BlockSpec((rows,), lambda i: (i,)),
        out_shape=jax.ShapeDtypeStruct((n,), jnp.float32),
    )(w, emb2)


def kernel(input_ids, embeddings, u_param):
    b, l = input_ids.shape
    d = embeddings.shape[-1]
    n = b * l
    idx = input_ids.reshape(n).astype(jnp.int32)
    emb2 = embeddings.reshape(n, d)
    w = _sc_gather(u_param.astype(jnp.float32), idx)
    out = _scale_rowsum_tc(w, emb2)
    return out.reshape(b, l)


# speedup vs baseline: 1.0777x; 1.0624x over previous
import functools

import jax
import jax.numpy as jnp
from jax import lax
from jax.experimental import pallas as pl
from jax.experimental.pallas import tpu as pltpu
from jax.experimental.pallas import tpu_sc as plsc

_NC = 2
_NS = 16
_NW = _NC * _NS
_LANES = 16


def _sc_gather(table, idx, chunk=6400, unroll=1):
    vocab = table.shape[0]
    n = idx.shape[0]
    n_per_w = n // _NW

    mesh = plsc.VectorSubcoreMesh(core_axis_name="c", subcore_axis_name="s")

    nch = n_per_w // chunk

    @functools.partial(
        pl.kernel,
        mesh=mesh,
        out_type=jax.ShapeDtypeStruct((n,), jnp.float32),
        scratch_types=[
            pltpu.VMEM((vocab,), jnp.float32),
            pltpu.VMEM((2, chunk), jnp.int32),
            pltpu.VMEM((2, chunk), jnp.float32),
            pltpu.SemaphoreType.DMA,
            pltpu.SemaphoreType.DMA,
            pltpu.SemaphoreType.DMA,
        ],
        compiler_params=pltpu.CompilerParams(needs_layout_passes=False),
    )
    def gather_kernel(table_hbm, idx_hbm, out_hbm, table_v, idx_v, w_v,
                      sem_t, sem_i, sem_o):
        wid = lax.axis_index("s") * _NC + lax.axis_index("c")
        base = wid * n_per_w

        t_copy = pltpu.async_copy(table_hbm, table_v, sem_t)
        h_idx = [None] * nch
        h_out = [None] * nch
        h_idx[0] = pltpu.async_copy(
            idx_hbm.at[pl.ds(base, chunk)], idx_v.at[0], sem_i)
        t_copy.wait()

        for ci in range(nch):
            buf = ci % 2
            off = base + ci * chunk
            if ci + 1 < nch:
                h_idx[ci + 1] = pltpu.async_copy(
                    idx_hbm.at[pl.ds(off + chunk, chunk)],
                    idx_v.at[1 - buf], sem_i)
            h_idx[ci].wait()
            if ci >= 2:
                h_out[ci - 2].wait()

            def vec_body(i, c2, buf=buf):
                o16 = pl.multiple_of(i * _LANES, _LANES)
                vidx = idx_v[buf, pl.ds(o16, _LANES)]
                w_v[buf, pl.ds(o16, _LANES)] = plsc.load_gather(
                    table_v, [vidx])
                return c2

            lax.fori_loop(0, chunk // _LANES, vec_body, 0,
                          unroll=max(unroll, 1))
            h_out[ci] = pltpu.async_copy(
                w_v.at[buf], out_hbm.at[pl.ds(off, chunk)], sem_o)

        for ci in range(max(nch - 2, 0), nch):
            h_out[ci].wait()

    return gather_kernel(table, idx)


def _rowsum_tc(emb2, rows=40960):
    n, d = emb2.shape

    def body(x_ref, o_ref):
        ones = jnp.ones((1, x_ref.shape[-1]), jnp.float32)
        s = jax.lax.dot_general(
            ones, x_ref[...],
            dimension_numbers=(((1,), (1,)), ((), ())),
            preferred_element_type=jnp.float32,
        )
        o_ref[...] = s[0, :]

    return pl.pallas_call(
        body,
        grid=(n // rows,),
        in_specs=[pl.BlockSpec((rows, d), lambda i: (i, 0))],
        out_specs=pl.BlockSpec((rows,), lambda i: (i,)),
        out_shape=jax.ShapeDtypeStruct((n,), jnp.float32),
        cost_estimate=pl.CostEstimate(
            flops=n * d, bytes_accessed=n * d * 4 + n * 4, transcendentals=0,
        ),
    )(emb2)


def _mul_tc(w, s, rows=102400):
    n = w.shape[0]

    def body(w_ref, s_ref, o_ref):
        o_ref[...] = w_ref[...] * s_ref[...]

    return pl.pallas_call(
        body,
        grid=(n // rows,),
        in_specs=[
            pl.BlockSpec((rows,), lambda i: (i,)),
            pl.BlockSpec((rows,), lambda i: (i,)),
        ],
        out_specs=pl.BlockSpec((rows,), lambda i: (i,)),
        out_shape=jax.ShapeDtypeStruct((n,), jnp.float32),
    )(w, s)


def kernel(input_ids, embeddings, u_param):
    b, l = input_ids.shape
    d = embeddings.shape[-1]
    n = b * l
    idx = input_ids.reshape(n).astype(jnp.int32)
    emb2 = embeddings.reshape(n, d)
    w = _sc_gather(u_param.astype(jnp.float32), idx)
    s = _rowsum_tc(emb2)
    out = _mul_tc(w, s)
    return out.reshape(b, l)


# speedup vs baseline: 1.0777x; 1.0001x over previous
import functools

import jax
import jax.numpy as jnp
from jax import lax
from jax.experimental import pallas as pl
from jax.experimental.pallas import tpu as pltpu
from jax.experimental.pallas import tpu_sc as plsc

_NC = 2
_NS = 16
_NW = _NC * _NS
_LANES = 16


def _sc_gather(table, idx, chunk=6400, unroll=1):
    vocab = table.shape[0]
    n = idx.shape[0]
    n_per_w = n // _NW

    mesh = plsc.VectorSubcoreMesh(core_axis_name="c", subcore_axis_name="s")

    nch = n_per_w // chunk

    @functools.partial(
        pl.kernel,
        mesh=mesh,
        out_type=jax.ShapeDtypeStruct((n,), jnp.float32),
        scratch_types=[
            pltpu.VMEM((vocab,), jnp.float32),
            pltpu.VMEM((2, chunk), jnp.int32),
            pltpu.VMEM((2, chunk), jnp.float32),
            pltpu.SemaphoreType.DMA,
            pltpu.SemaphoreType.DMA,
            pltpu.SemaphoreType.DMA,
        ],
        compiler_params=pltpu.CompilerParams(needs_layout_passes=False),
    )
    def gather_kernel(table_hbm, idx_hbm, out_hbm, table_v, idx_v, w_v,
                      sem_t, sem_i, sem_o):
        wid = lax.axis_index("s") * _NC + lax.axis_index("c")
        base = wid * n_per_w

        t_copy = pltpu.async_copy(table_hbm, table_v, sem_t)
        h_idx = [None] * nch
        h_out = [None] * nch
        h_idx[0] = pltpu.async_copy(
            idx_hbm.at[pl.ds(base, chunk)], idx_v.at[0], sem_i)
        t_copy.wait()

        for ci in range(nch):
            buf = ci % 2
            off = base + ci * chunk
            if ci + 1 < nch:
                h_idx[ci + 1] = pltpu.async_copy(
                    idx_hbm.at[pl.ds(off + chunk, chunk)],
                    idx_v.at[1 - buf], sem_i)
            h_idx[ci].wait()
            if ci >= 2:
                h_out[ci - 2].wait()

            def vec_body(i, c2, buf=buf):
                o16 = pl.multiple_of(i * _LANES, _LANES)
                vidx = idx_v[buf, pl.ds(o16, _LANES)]
                w_v[buf, pl.ds(o16, _LANES)] = plsc.load_gather(
                    table_v, [vidx])
                return c2

            lax.fori_loop(0, chunk // _LANES, vec_body, 0,
                          unroll=max(unroll, 1))
            h_out[ci] = pltpu.async_copy(
                w_v.at[buf], out_hbm.at[pl.ds(off, chunk)], sem_o)

        for ci in range(max(nch - 2, 0), nch):
            h_out[ci].wait()

    return gather_kernel(table, idx)


def _rowsum_tc(emb2, rows=40960):
    n, d = emb2.shape

    def body(x_ref, o_ref):
        ones = jnp.ones((1, x_ref.shape[-1]), jnp.float32)
        s = jax.lax.dot_general(
            ones, x_ref[...],
            dimension_numbers=(((1,), (1,)), ((), ())),
            preferred_element_type=jnp.float32,
        )
        o_ref[...] = s[0, :]

    return pl.pallas_call(
        body,
        grid=(n // rows,),
        in_specs=[pl.BlockSpec((rows, d), lambda i: (i, 0))],
        out_specs=pl.BlockSpec((rows,), lambda i: (i,)),
        out_shape=jax.ShapeDtypeStruct((n,), jnp.float32),
        cost_estimate=pl.CostEstimate(
            flops=n * d, bytes_accessed=n * d * 4 + n * 4, transcendentals=0,
        ),
    )(emb2)


def _mul_tc(w, s, rows=102400):
    n = w.shape[0]

    def body(w_ref, s_ref, o_ref):
        o_ref[...] = w_ref[...] * s_ref[...]

    return pl.pallas_call(
        body,
        grid=(n // rows,),
        in_specs=[
            pl.BlockSpec((rows,), lambda i: (i,)),
            pl.BlockSpec((rows,), lambda i: (i,)),
        ],
        out_specs=pl.BlockSpec((rows,), lambda i: (i,)),
        out_shape=jax.ShapeDtypeStruct((n,), jnp.float32),
    )(w, s)


def kernel(input_ids, embeddings, u_param):
    b, l = input_ids.shape
    d = embeddings.shape[-1]
    n = b * l
    idx = input_ids.reshape(n).astype(jnp.int32)
    emb2 = embeddings.reshape(n, d)
    s = _rowsum_tc(emb2)
    w = _sc_gather(u_param.astype(jnp.float32), idx)
    out = _mul_tc(w, s)
    return out.reshape(b, l)
